# baseline (speedup 1.0000x reference)
import functools

import jax
import jax.numpy as jnp
from jax import lax
from jax.experimental import pallas as pl
from jax.experimental.pallas import tpu as pltpu
from jax.experimental.pallas import tpu_sc as plsc

N = 10000
IN = 128
HID = 256
OUT = 128
H = 4
C = HID // H

NT = 10016
NPAD = 10240
KE = 96
NSUB = 16
NCORE = 2
NCH = 216
PER_TEC = NCH * KE
EPAD = PER_TEC * NSUB
NG = KE // 16
BLK = 400
DUMMY = N


def _pre_body(x_ref, wint_ref, bin_ref, wbut_ref, asb_ref, adb_ref,
              wtdt_ref, ast_ref, adt_ref, m_ref,
              h2bu_ref, h2td_ref, abu_d_ref, atd_d_ref):
    h = jnp.maximum(
        jnp.dot(x_ref[...], wint_ref[...], precision="highest") + bin_ref[...],
        0.0)
    m = m_ref[...]
    z30 = jnp.zeros((h.shape[0], 30), jnp.float32)
    for (wt, a_s, a_d, h2_ref, d_ref) in (
        (wbut_ref, asb_ref, adb_ref, h2bu_ref, abu_d_ref),
        (wtdt_ref, ast_ref, adt_ref, h2td_ref, atd_d_ref),
    ):
        h2 = jnp.dot(h, wt[...], precision="highest")
        a_src = jnp.dot(h2 * a_s[...], m, precision="highest")
        a_dst = jnp.dot(h2 * a_d[...], m, precision="highest")
        h2_ref[0] = jnp.concatenate(
            [h2[:, :128], a_src[:, 0:2], z30], axis=-1).astype(jnp.bfloat16)
        h2_ref[1] = jnp.concatenate(
            [h2[:, 128:], a_src[:, 2:4], z30], axis=-1).astype(jnp.bfloat16)
        d_ref[0] = jnp.concatenate([a_dst[:, 0:2], z30[:, :14]], axis=-1)
        d_ref[1] = jnp.concatenate([a_dst[:, 2:4], z30[:, :14]], axis=-1)


def _run_pre(x, WinT, b_in, WbuT, asb, adb, WtdT, ast, adt, m):
    grid = (N // BLK,)
    wspec = lambda shape: pl.BlockSpec(shape, lambda i: tuple(0 for _ in shape))
    return pl.pallas_call(
        _pre_body,
        grid=grid,
        in_specs=[
            pl.BlockSpec((BLK, IN), lambda i: (i, 0)),
            wspec((IN, HID)), wspec((HID,)),
            wspec((HID, HID)), wspec((HID,)), wspec((HID,)),
            wspec((HID, HID)), wspec((HID,)), wspec((HID,)),
            wspec((HID, H)),
        ],
        out_specs=[
            pl.BlockSpec((NCORE, BLK, 160), lambda i: (0, i, 0)),
            pl.BlockSpec((NCORE, BLK, 160), lambda i: (0, i, 0)),
            pl.BlockSpec((NCORE, BLK, 16), lambda i: (0, i, 0)),
            pl.BlockSpec((NCORE, BLK, 16), lambda i: (0, i, 0)),
        ],
        out_shape=[
            jax.ShapeDtypeStruct((NCORE, N, 160), jnp.bfloat16),
            jax.ShapeDtypeStruct((NCORE, N, 160), jnp.bfloat16),
            jax.ShapeDtypeStruct((NCORE, NT, 16), jnp.float32),
            jax.ShapeDtypeStruct((NCORE, NT, 16), jnp.float32),
        ],
    )(x, WinT, b_in, WbuT, asb, adb, WtdT, ast, adt, m)


def _sc_body(h2cat, adst, edges_pk, out_msg,
             acc_msg, ed, gidx, didx, dstv, sidx, rows, adr, msg,
             sem_i0, sem_i1, sem_g0, sem_g1, sem_a0, sem_a1, sem_s):
    c = lax.axis_index("c")
    s = lax.axis_index("s")
    sem_i = (sem_i0, sem_i1)
    sem_g = (sem_g0, sem_g1)
    sem_a = (sem_a0, sem_a1)

    def zrow(i, _):
        for j in range(9):
            msg[i, pl.ds(j * 16, 16)] = jnp.zeros((16,), jnp.float32)
        return 0

    lax.fori_loop(0, KE, zrow, 0)
    rows_per_tile = NPAD // NSUB
    for k in range(rows_per_tile // KE):
        pltpu.sync_copy(msg, acc_msg.at[pl.ds(s * rows_per_tile + k * KE, KE)])
    rem = rows_per_tile - (rows_per_tile // KE) * KE
    if rem:
        pltpu.sync_copy(msg.at[pl.ds(0, rem)],
                        acc_msg.at[pl.ds((s + 1) * rows_per_tile - rem, rem)])
    plsc.subcore_barrier()

    def start_idx(t, b):
        pltpu.async_copy(edges_pk.at[s, t], ed.at[b], sem_i[b])

    def mid(t, b):
        pltpu.make_async_copy(edges_pk.at[s, t], ed.at[b], sem_i[b]).wait()

        def mkidx(g, _):
            sv = ed[b, pl.ds(g * 16, 16)]
            dv = ed[b, pl.ds(KE + g * 16, 16)]
            gidx[b, pl.ds(g * 16, 16)] = sv + c * N
            didx[b, pl.ds(g * 16, 16)] = dv + c * NT
            dstv[b, g, pl.ds(0, 16)] = dv
            return 0

        lax.fori_loop(0, NG, mkidx, 0)
        pltpu.async_copy(h2cat.at[gidx.at[b]], rows.at[b], sem_g[b])
        pltpu.async_copy(adst.at[didx.at[b]], adr.at[b], sem_a[b])

    def wait_scatters():
        for g in range(NG):
            pltpu.make_async_copy(msg.at[pl.ds(g * 16, 16)],
                                  acc_msg.at[sidx.at[g]], sem_s).wait()

    def finish(t, b):
        lane = lax.iota(jnp.int32, 16)
        pltpu.make_async_copy(h2cat.at[gidx.at[b]], rows.at[b], sem_g[b]).wait()
        pltpu.make_async_copy(adst.at[didx.at[b]], adr.at[b], sem_a[b]).wait()

        @pl.when(t > 0)
        def _():
            wait_scatters()

        def grp(g, _):
            rowi = g * 16 + lane
            z16 = jnp.zeros((16,), jnp.int32)
            w = plsc.load_gather(rows.at[b], [rowi, z16 + 64])
            asp = plsc.unpack(plsc.bitcast(w, jnp.bfloat16),
                              format=plsc.PackFormat.INTERLEAVED)
            a0 = asp[0] + plsc.load_gather(adr.at[b], [rowi, z16])
            a1 = asp[1] + plsc.load_gather(adr.at[b], [rowi, z16 + 1])
            a0 = jnp.where(a0 >= 0.0, a0, 0.2 * a0)
            a1 = jnp.where(a1 >= 0.0, a1, 0.2 * a1)
            e0 = jnp.exp(a0)
            e1 = jnp.exp(a1)
            sidx[g, pl.ds(0, 16)] = dstv[b, g, pl.ds(0, 16)]
            plsc.store_scatter(msg, [rowi, z16 + 130], e0)
            plsc.store_scatter(msg, [rowi, z16 + 131], e1)
            for lj in range(16):
                s0 = e0[lj]
                s1 = e1[lj]
                e = g * 16 + lj
                for j in range(4):
                    w4 = rows[b, e, pl.ds(j * 16, 16)]
                    pq = plsc.unpack(plsc.bitcast(w4, jnp.bfloat16),
                                     format=plsc.PackFormat.INTERLEAVED)
                    sc = s0 if j < 2 else s1
                    msg[e, pl.ds(j * 32, 16)] = sc * pq[0]
                    msg[e, pl.ds(j * 32 + 16, 16)] = sc * pq[1]
            pltpu.async_copy(msg.at[pl.ds(g * 16, 16)],
                             acc_msg.at[sidx.at[g]], sem_s, add=True)
            return 0

        lax.fori_loop(0, NG, grp, 0)

    start_idx(0, 0)
    start_idx(1, 1)
    mid(0, 0)
    mid(1, 1)

    def pipe(k, _):
        t0 = 2 * k

        @pl.when(t0 + 2 < NCH)
        def _():
            start_idx(t0 + 2, 0)
            start_idx(t0 + 3, 1)

        finish(t0, 0)

        @pl.when(t0 + 2 < NCH)
        def _():
            mid(t0 + 2, 0)

        finish(t0 + 1, 1)

        @pl.when(t0 + 3 < NCH)
        def _():
            mid(t0 + 3, 1)

        return 0

    lax.fori_loop(0, NCH // 2, pipe, 0)
    wait_scatters()
    plsc.subcore_barrier()

    pltpu.sync_copy(acc_msg.at[pl.ds(s * rows_per_tile, rows_per_tile)],
                    out_msg.at[c, pl.ds(s * rows_per_tile, rows_per_tile)])
    return


def _run_gat_sc(h2cat, adst, edges_pk):
    mesh = plsc.VectorSubcoreMesh(core_axis_name="c", subcore_axis_name="s",
                                  num_cores=NCORE, num_subcores=NSUB)
    f = pl.kernel(
        _sc_body,
        out_type=[
            jax.ShapeDtypeStruct((NCORE, NPAD, 144), jnp.float32),
        ],
        mesh=mesh,
        compiler_params=pltpu.CompilerParams(needs_layout_passes=False,
                                             use_tc_tiling_on_sc=False),
        scratch_types=[
            pltpu.VMEM_SHARED((NPAD, 144), jnp.float32),
            pltpu.VMEM((2, 2 * KE), jnp.int32),
            pltpu.VMEM((2, KE), jnp.int32),
            pltpu.VMEM((2, KE), jnp.int32),
            pltpu.VMEM((2, NG, 16), jnp.int32),
            pltpu.VMEM((NG, 16), jnp.int32),
            pltpu.VMEM((2, KE, 80), jnp.int32),
            pltpu.VMEM((2, KE, 16), jnp.float32),
            pltpu.VMEM((KE, 144), jnp.float32),
            pltpu.SemaphoreType.DMA,
            pltpu.SemaphoreType.DMA,
            pltpu.SemaphoreType.DMA,
            pltpu.SemaphoreType.DMA,
            pltpu.SemaphoreType.DMA,
            pltpu.SemaphoreType.DMA,
            pltpu.SemaphoreType.DMA,
        ],
    )
    (out,) = f(h2cat, adst, edges_pk)
    return out


def _ln(y, g, b):
    mu = jnp.mean(y, axis=-1, keepdims=True)
    var = jnp.mean((y - mu) ** 2, axis=-1, keepdims=True)
    return (y - mu) / jnp.sqrt(var + 1e-5) * g + b


def _post_body(mbu_ref, mtd_ref, bbu_ref, gbu_ref, bebu_ref,
               btd_ref, gtd_ref, betd_ref, wfut_ref, bfu_ref, go_ref, beo_ref,
               o_ref):
    outs = []
    for m_ref, b_ref, g_ref, be_ref in (
        (mbu_ref, bbu_ref, gbu_ref, bebu_ref),
        (mtd_ref, btd_ref, gtd_ref, betd_ref),
    ):
        cols = []
        for core in range(NCORE):
            blkc = m_ref[core]
            for hh in range(2):
                d1 = blkc[:, 130 + hh:131 + hh] + 1e-16
                cols.append(blkc[:, hh * 64:(hh + 1) * 64] / d1)
        y = jnp.concatenate(cols, axis=-1) + b_ref[...]
        outs.append(jnp.maximum(_ln(y, g_ref[...], be_ref[...]), 0.0))
    fused = jnp.concatenate(outs, axis=-1)
    y = jnp.dot(fused, wfut_ref[...], precision="highest") + bfu_ref[...]
    y = _ln(y, go_ref[...], beo_ref[...])
    nrm = jnp.sqrt(jnp.sum(y * y, axis=-1, keepdims=True))
    o_ref[...] = y / jnp.maximum(nrm, 1e-12)


def _run_post(mbu, mtd, b_bu, g_bu, be_bu, b_td, g_td, be_td,
              WfuT, b_fu, g_out, be_out):
    grid = (N // BLK,)
    wspec = lambda shape: pl.BlockSpec(shape, lambda i: tuple(0 for _ in shape))
    mspec = pl.BlockSpec((NCORE, BLK, 144), lambda i: (0, i, 0))
    return pl.pallas_call(
        _post_body,
        grid=grid,
        in_specs=[
            mspec, mspec,
            wspec((HID,)), wspec((HID,)), wspec((HID,)),
            wspec((HID,)), wspec((HID,)), wspec((HID,)),
            wspec((2 * HID, OUT)), wspec((OUT,)), wspec((OUT,)), wspec((OUT,)),
        ],
        out_specs=pl.BlockSpec((BLK, OUT), lambda i: (i, 0)),
        out_shape=jax.ShapeDtypeStruct((N, OUT), jnp.float32),
    )(mbu, mtd, b_bu, g_bu, be_bu, b_td, g_td, be_td,
      WfuT, b_fu, g_out, be_out)


def _pad_edges(edge_index):
    npad = EPAD - (edge_index.shape[1] + N)
    loop = jnp.arange(N, dtype=jnp.int32)
    src = jnp.concatenate([edge_index[0], loop,
                           jnp.zeros((npad,), jnp.int32)])
    dst = jnp.concatenate([edge_index[1], loop,
                           jnp.full((npad,), DUMMY, jnp.int32)])
    srcr = src.reshape(NSUB, NCH, 1, KE)
    dstr = dst.reshape(NSUB, NCH, 1, KE)
    return jnp.concatenate([srcr, dstr], axis=2).reshape(NSUB, NCH, 2 * KE)


def _bf16_perm():
    i = jnp.arange(2 * HID)
    blk, off = i // 32, i % 32
    return blk * 32 + jnp.where(off < 16, 2 * off, 2 * (off - 16) + 1)


def kernel(x, edge_index_bu, edge_index_td, W_in, b_in, W_bu, att_src_bu,
           att_dst_bu, b_bu, g_bu, be_bu, W_td, att_src_td, att_dst_td, b_td,
           g_td, be_td, W_fu, b_fu, g_out, be_out):
    m = jnp.repeat(jnp.eye(H, dtype=jnp.float32), C, axis=0)
    h2bu, h2td, abu_d, atd_d = _run_pre(
        x, W_in.T, b_in, W_bu.T, att_src_bu.reshape(-1), att_dst_bu.reshape(-1),
        W_td.T, att_src_td.reshape(-1), att_dst_td.reshape(-1), m)

    epk_bu = _pad_edges(edge_index_bu)
    epk_td = _pad_edges(edge_index_td)

    def as_i32(slab):
        return lax.bitcast_convert_type(
            slab.reshape(NCORE, N, 80, 2), jnp.int32).reshape(NCORE * N, 80)

    mbu = _run_gat_sc(as_i32(h2bu), abu_d.reshape(NCORE * NT, 16), epk_bu)
    mtd = _run_gat_sc(as_i32(h2td), atd_d.reshape(NCORE * NT, 16), epk_td)

    perm = _bf16_perm()
    p256 = perm[:HID]
    p512 = jnp.concatenate([p256, HID + p256])
    return _run_post(mbu, mtd, b_bu[p256], g_bu[p256], be_bu[p256],
                     b_td[p256], g_td[p256], be_td[p256],
                     W_fu.T[p512], b_fu, g_out, be_out)


# speedup vs baseline: 1.4369x; 1.4369x over previous
import functools

import jax
import jax.numpy as jnp
from jax import lax
from jax.experimental import pallas as pl
from jax.experimental.pallas import tpu as pltpu
from jax.experimental.pallas import tpu_sc as plsc

N = 10000
IN = 128
HID = 256
OUT = 128
H = 4
C = HID // H

NT = 10016
NPAD = 10240
KE = 112
NSUB = 16
NCORE = 2
NCH = 186
PER_TEC = NCH * KE
EPAD = PER_TEC * NSUB
NG = KE // 16
BLK = 400
DUMMY = N


def _pre_body(x_ref, wint_ref, bin_ref, wbut_ref, asb_ref, adb_ref,
              wtdt_ref, ast_ref, adt_ref, m_ref,
              h2bu_ref, h2td_ref, abu_d_ref, atd_d_ref):
    h = jnp.maximum(
        jnp.dot(x_ref[...], wint_ref[...]) + bin_ref[...],
        0.0)
    m = m_ref[...]
    z14 = jnp.zeros((h.shape[0], 14), jnp.float32)
    for (wt, a_s, a_d, h2_ref, d_ref) in (
        (wbut_ref, asb_ref, adb_ref, h2bu_ref, abu_d_ref),
        (wtdt_ref, ast_ref, adt_ref, h2td_ref, atd_d_ref),
    ):
        h2 = jnp.dot(h, wt[...])
        a_src = jnp.dot(h2 * a_s[...], m)
        a_dst = jnp.dot(h2 * a_d[...], m)
        h2_ref[0] = jnp.concatenate([h2[:, :128], a_src[:, 0:2], z14], axis=-1)
        h2_ref[1] = jnp.concatenate([h2[:, 128:], a_src[:, 2:4], z14], axis=-1)
        d_ref[0] = jnp.concatenate([a_dst[:, 0:2], z14], axis=-1)
        d_ref[1] = jnp.concatenate([a_dst[:, 2:4], z14], axis=-1)


def _run_pre(x, WinT, b_in, WbuT, asb, adb, WtdT, ast, adt, m):
    grid = (N // BLK,)
    wspec = lambda shape: pl.BlockSpec(shape, lambda i: tuple(0 for _ in shape))
    return pl.pallas_call(
        _pre_body,
        grid=grid,
        in_specs=[
            pl.BlockSpec((BLK, IN), lambda i: (i, 0)),
            wspec((IN, HID)), wspec((HID,)),
            wspec((HID, HID)), wspec((HID,)), wspec((HID,)),
            wspec((HID, HID)), wspec((HID,)), wspec((HID,)),
            wspec((HID, H)),
        ],
        out_specs=[
            pl.BlockSpec((NCORE, BLK, 144), lambda i: (0, i, 0)),
            pl.BlockSpec((NCORE, BLK, 144), lambda i: (0, i, 0)),
            pl.BlockSpec((NCORE, BLK, 16), lambda i: (0, i, 0)),
            pl.BlockSpec((NCORE, BLK, 16), lambda i: (0, i, 0)),
        ],
        out_shape=[
            jax.ShapeDtypeStruct((NCORE, N, 144), jnp.float32),
            jax.ShapeDtypeStruct((NCORE, N, 144), jnp.float32),
            jax.ShapeDtypeStruct((NCORE, NT, 16), jnp.float32),
            jax.ShapeDtypeStruct((NCORE, NT, 16), jnp.float32),
        ],
    )(x, WinT, b_in, WbuT, asb, adb, WtdT, ast, adt, m)


def _sc_body(h2cat, adst, edges_pk, out_msg,
             acc_msg, ed, gidx, didx, dstv, rows, adr,
             sem_i0, sem_i1, sem_g0, sem_g1, sem_a0, sem_a1, sem_s0, sem_s1):
    c = lax.axis_index("c")
    s = lax.axis_index("s")
    sem_i = (sem_i0, sem_i1)
    sem_g = (sem_g0, sem_g1)
    sem_a = (sem_a0, sem_a1)
    sem_s = (sem_s0, sem_s1)

    def zrow(i, _):
        for j in range(9):
            rows[0, i, pl.ds(j * 16, 16)] = jnp.zeros((16,), jnp.float32)
        return 0

    lax.fori_loop(0, KE, zrow, 0)
    rows_per_tile = NPAD // NSUB
    zsrc = rows.at[0]
    for k in range(rows_per_tile // KE):
        pltpu.sync_copy(zsrc, acc_msg.at[pl.ds(s * rows_per_tile + k * KE, KE)])
    rem = rows_per_tile - (rows_per_tile // KE) * KE
    if rem:
        pltpu.sync_copy(zsrc.at[pl.ds(0, rem)],
                        acc_msg.at[pl.ds((s + 1) * rows_per_tile - rem, rem)])
    plsc.subcore_barrier()

    def start_idx(t, b):
        pltpu.async_copy(edges_pk.at[s, t], ed.at[b], sem_i[b])

    def wait_scatters(b):
        for g in range(NG):
            pltpu.make_async_copy(rows.at[b, pl.ds(g * 16, 16)],
                                  acc_msg.at[dstv.at[b, g]], sem_s[b]).wait()

    def mid(t, b, wait_scatter):
        if wait_scatter:
            wait_scatters(b)
        pltpu.make_async_copy(edges_pk.at[s, t], ed.at[b], sem_i[b]).wait()

        def mkidx(g, _):
            sv = ed[b, pl.ds(g * 16, 16)]
            dv = ed[b, pl.ds(KE + g * 16, 16)]
            gidx[b, pl.ds(g * 16, 16)] = sv + c * N
            didx[b, pl.ds(g * 16, 16)] = dv + c * NT
            dstv[b, g, pl.ds(0, 16)] = dv
            return 0

        lax.fori_loop(0, NG, mkidx, 0)
        pltpu.async_copy(h2cat.at[gidx.at[b]], rows.at[b], sem_g[b])
        pltpu.async_copy(adst.at[didx.at[b]], adr.at[b], sem_a[b])

    def finish(t, b):
        pltpu.make_async_copy(h2cat.at[gidx.at[b]], rows.at[b], sem_g[b]).wait()
        pltpu.make_async_copy(adst.at[didx.at[b]], adr.at[b], sem_a[b]).wait()
        lane = lax.iota(jnp.int32, 16)

        def grp(g, _):
            rowi = g * 16 + lane
            c128 = jnp.full((16,), 128, jnp.int32)
            a0 = (plsc.load_gather(rows.at[b], [rowi, c128])
                  + plsc.load_gather(adr.at[b], [rowi, c128 - 128]))
            a1 = (plsc.load_gather(rows.at[b], [rowi, c128 + 1])
                  + plsc.load_gather(adr.at[b], [rowi, c128 - 127]))
            a0 = jnp.where(a0 >= 0.0, a0, 0.2 * a0)
            a1 = jnp.where(a1 >= 0.0, a1, 0.2 * a1)
            e0 = jnp.exp(a0)
            e1 = jnp.exp(a1)
            plsc.store_scatter(rows.at[b], [rowi, c128 + 2], e0)
            plsc.store_scatter(rows.at[b], [rowi, c128 + 3], e1)
            for lj in range(16):
                s0 = e0[lj]
                s1 = e1[lj]
                e = g * 16 + lj
                for j in range(4):
                    rows[b, e, pl.ds(j * 16, 16)] = (
                        s0 * rows[b, e, pl.ds(j * 16, 16)])
                    o = 64 + j * 16
                    rows[b, e, pl.ds(o, 16)] = s1 * rows[b, e, pl.ds(o, 16)]
            pltpu.async_copy(rows.at[b, pl.ds(g * 16, 16)],
                             acc_msg.at[dstv.at[b, g]], sem_s[b], add=True)
            return 0

        lax.fori_loop(0, NG, grp, 0)

    start_idx(0, 0)
    start_idx(1, 1)
    mid(0, 0, wait_scatter=False)
    mid(1, 1, wait_scatter=False)

    def pipe(k, _):
        t0 = 2 * k

        @pl.when(t0 + 2 < NCH)
        def _():
            start_idx(t0 + 2, 0)
            start_idx(t0 + 3, 1)

        finish(t0, 0)

        @pl.when(t0 + 2 < NCH)
        def _():
            mid(t0 + 2, 0, wait_scatter=True)

        finish(t0 + 1, 1)

        @pl.when(t0 + 3 < NCH)
        def _():
            mid(t0 + 3, 1, wait_scatter=True)

        return 0

    lax.fori_loop(0, NCH // 2, pipe, 0)
    wait_scatters(0)
    wait_scatters(1)
    plsc.subcore_barrier()

    pltpu.sync_copy(acc_msg.at[pl.ds(s * rows_per_tile, rows_per_tile)],
                    out_msg.at[c, pl.ds(s * rows_per_tile, rows_per_tile)])
    return


def _run_gat_sc(h2cat, adst, edges_pk):
    mesh = plsc.VectorSubcoreMesh(core_axis_name="c", subcore_axis_name="s",
                                  num_cores=NCORE, num_subcores=NSUB)
    f = pl.kernel(
        _sc_body,
        out_type=[
            jax.ShapeDtypeStruct((NCORE, NPAD, 144), jnp.float32),
        ],
        mesh=mesh,
        compiler_params=pltpu.CompilerParams(needs_layout_passes=False,
                                             use_tc_tiling_on_sc=False),
        scratch_types=[
            pltpu.VMEM_SHARED((NPAD, 144), jnp.float32),
            pltpu.VMEM((2, 2 * KE), jnp.int32),
            pltpu.VMEM((2, KE), jnp.int32),
            pltpu.VMEM((2, KE), jnp.int32),
            pltpu.VMEM((2, NG, 16), jnp.int32),
            pltpu.VMEM((2, KE, 144), jnp.float32),
            pltpu.VMEM((2, KE, 16), jnp.float32),
            pltpu.SemaphoreType.DMA,
            pltpu.SemaphoreType.DMA,
            pltpu.SemaphoreType.DMA,
            pltpu.SemaphoreType.DMA,
            pltpu.SemaphoreType.DMA,
            pltpu.SemaphoreType.DMA,
            pltpu.SemaphoreType.DMA,
            pltpu.SemaphoreType.DMA,
        ],
    )
    (out,) = f(h2cat, adst, edges_pk)
    return out


def _ln(y, g, b):
    mu = jnp.mean(y, axis=-1, keepdims=True)
    var = jnp.mean((y - mu) ** 2, axis=-1, keepdims=True)
    return (y - mu) / jnp.sqrt(var + 1e-5) * g + b


def _post_body(mbu_ref, mtd_ref, bbu_ref, gbu_ref, bebu_ref,
               btd_ref, gtd_ref, betd_ref, wfut_ref, bfu_ref, go_ref, beo_ref,
               o_ref):
    outs = []
    for m_ref, b_ref, g_ref, be_ref in (
        (mbu_ref, bbu_ref, gbu_ref, bebu_ref),
        (mtd_ref, btd_ref, gtd_ref, betd_ref),
    ):
        cols = []
        for core in range(NCORE):
            blkc = m_ref[core]
            for hh in range(2):
                d1 = blkc[:, 130 + hh:131 + hh] + 1e-16
                cols.append(blkc[:, hh * 64:(hh + 1) * 64] / d1)
        y = jnp.concatenate(cols, axis=-1) + b_ref[...]
        outs.append(jnp.maximum(_ln(y, g_ref[...], be_ref[...]), 0.0))
    fused = jnp.concatenate(outs, axis=-1)
    y = jnp.dot(fused, wfut_ref[...]) + bfu_ref[...]
    y = _ln(y, go_ref[...], beo_ref[...])
    nrm = jnp.sqrt(jnp.sum(y * y, axis=-1, keepdims=True))
    o_ref[...] = y / jnp.maximum(nrm, 1e-12)


def _run_post(mbu, mtd, b_bu, g_bu, be_bu, b_td, g_td, be_td,
              WfuT, b_fu, g_out, be_out):
    grid = (N // BLK,)
    wspec = lambda shape: pl.BlockSpec(shape, lambda i: tuple(0 for _ in shape))
    mspec = pl.BlockSpec((NCORE, BLK, 144), lambda i: (0, i, 0))
    return pl.pallas_call(
        _post_body,
        grid=grid,
        in_specs=[
            mspec, mspec,
            wspec((HID,)), wspec((HID,)), wspec((HID,)),
            wspec((HID,)), wspec((HID,)), wspec((HID,)),
            wspec((2 * HID, OUT)), wspec((OUT,)), wspec((OUT,)), wspec((OUT,)),
        ],
        out_specs=pl.BlockSpec((BLK, OUT), lambda i: (i, 0)),
        out_shape=jax.ShapeDtypeStruct((N, OUT), jnp.float32),
    )(mbu, mtd, b_bu, g_bu, be_bu, b_td, g_td, be_td,
      WfuT, b_fu, g_out, be_out)


def _pad_edges(edge_index):
    npad = EPAD - (edge_index.shape[1] + N)
    loop = jnp.arange(N, dtype=jnp.int32)
    src = jnp.concatenate([edge_index[0], loop,
                           jnp.zeros((npad,), jnp.int32)])
    dst = jnp.concatenate([edge_index[1], loop,
                           jnp.full((npad,), DUMMY, jnp.int32)])
    srcr = src.reshape(NSUB, NCH, 1, KE)
    dstr = dst.reshape(NSUB, NCH, 1, KE)
    return jnp.concatenate([srcr, dstr], axis=2).reshape(NSUB, NCH, 2 * KE)


def kernel(x, edge_index_bu, edge_index_td, W_in, b_in, W_bu, att_src_bu,
           att_dst_bu, b_bu, g_bu, be_bu, W_td, att_src_td, att_dst_td, b_td,
           g_td, be_td, W_fu, b_fu, g_out, be_out):
    m = jnp.repeat(jnp.eye(H, dtype=jnp.float32), C, axis=0)
    h2bu, h2td, abu_d, atd_d = _run_pre(
        x, W_in.T, b_in, W_bu.T, att_src_bu.reshape(-1), att_dst_bu.reshape(-1),
        W_td.T, att_src_td.reshape(-1), att_dst_td.reshape(-1), m)

    epk_bu = _pad_edges(edge_index_bu)
    epk_td = _pad_edges(edge_index_td)

    mbu = _run_gat_sc(h2bu.reshape(NCORE * N, 144),
                      abu_d.reshape(NCORE * NT, 16), epk_bu)
    mtd = _run_gat_sc(h2td.reshape(NCORE * N, 144),
                      atd_d.reshape(NCORE * NT, 16), epk_td)

    return _run_post(mbu, mtd, b_bu, g_bu, be_bu, b_td, g_td, be_td,
                     W_fu.T, b_fu, g_out, be_out)


# speedup vs baseline: 1.4695x; 1.0227x over previous
import functools

import jax
import jax.numpy as jnp
from jax import lax
from jax.experimental import pallas as pl
from jax.experimental.pallas import tpu as pltpu
from jax.experimental.pallas import tpu_sc as plsc

N = 10000
IN = 128
HID = 256
OUT = 128
H = 4
C = HID // H

NT = 10016
NPAD = 10240
KE = 112
NSUB = 16
NCORE = 2
NCH = 186
PER_TEC = NCH * KE
EPAD = PER_TEC * NSUB
NG = KE // 16
BLK = 1000
DUMMY = N


def _pre_body(x_ref, wint_ref, bin_ref, wbut_ref, asb_ref, adb_ref,
              wtdt_ref, ast_ref, adt_ref, m_ref,
              h2bu_ref, h2td_ref, abu_d_ref, atd_d_ref):
    h = jnp.maximum(
        jnp.dot(x_ref[...], wint_ref[...]) + bin_ref[...],
        0.0)
    m = m_ref[...]
    z14 = jnp.zeros((h.shape[0], 14), jnp.float32)
    for (wt, a_s, a_d, h2_ref, d_ref) in (
        (wbut_ref, asb_ref, adb_ref, h2bu_ref, abu_d_ref),
        (wtdt_ref, ast_ref, adt_ref, h2td_ref, atd_d_ref),
    ):
        h2 = jnp.dot(h, wt[...])
        a_src = jnp.dot(h2 * a_s[...], m)
        a_dst = jnp.dot(h2 * a_d[...], m)
        h2_ref[0] = jnp.concatenate([h2[:, :128], a_src[:, 0:2], z14], axis=-1)
        h2_ref[1] = jnp.concatenate([h2[:, 128:], a_src[:, 2:4], z14], axis=-1)
        d_ref[0] = jnp.concatenate([a_dst[:, 0:2], z14], axis=-1)
        d_ref[1] = jnp.concatenate([a_dst[:, 2:4], z14], axis=-1)


def _run_pre(x, WinT, b_in, WbuT, asb, adb, WtdT, ast, adt, m):
    grid = (N // BLK,)
    wspec = lambda shape: pl.BlockSpec(shape, lambda i: tuple(0 for _ in shape))
    return pl.pallas_call(
        _pre_body,
        grid=grid,
        in_specs=[
            pl.BlockSpec((BLK, IN), lambda i: (i, 0)),
            wspec((IN, HID)), wspec((HID,)),
            wspec((HID, HID)), wspec((HID,)), wspec((HID,)),
            wspec((HID, HID)), wspec((HID,)), wspec((HID,)),
            wspec((HID, H)),
        ],
        out_specs=[
            pl.BlockSpec((NCORE, BLK, 144), lambda i: (0, i, 0)),
            pl.BlockSpec((NCORE, BLK, 144), lambda i: (0, i, 0)),
            pl.BlockSpec((NCORE, BLK, 16), lambda i: (0, i, 0)),
            pl.BlockSpec((NCORE, BLK, 16), lambda i: (0, i, 0)),
        ],
        out_shape=[
            jax.ShapeDtypeStruct((NCORE, N, 144), jnp.float32),
            jax.ShapeDtypeStruct((NCORE, N, 144), jnp.float32),
            jax.ShapeDtypeStruct((NCORE, NT, 16), jnp.float32),
            jax.ShapeDtypeStruct((NCORE, NT, 16), jnp.float32),
        ],
    )(x, WinT, b_in, WbuT, asb, adb, WtdT, ast, adt, m)


def _sc_body(h2cat, adst, edges_pk, out_msg,
             acc_msg, ed, gidx, didx, dstv, rows, adr,
             sem_i0, sem_i1, sem_g0, sem_g1, sem_a0, sem_a1, sem_s0, sem_s1):
    c = lax.axis_index("c")
    s = lax.axis_index("s")
    sem_i = (sem_i0, sem_i1)
    sem_g = (sem_g0, sem_g1)
    sem_a = (sem_a0, sem_a1)
    sem_s = (sem_s0, sem_s1)

    def zrow(i, _):
        for j in range(9):
            rows[0, i, pl.ds(j * 16, 16)] = jnp.zeros((16,), jnp.float32)
        return 0

    lax.fori_loop(0, KE, zrow, 0)
    rows_per_tile = NPAD // NSUB
    zsrc = rows.at[0]
    for k in range(rows_per_tile // KE):
        pltpu.sync_copy(zsrc, acc_msg.at[pl.ds(s * rows_per_tile + k * KE, KE)])
    rem = rows_per_tile - (rows_per_tile // KE) * KE
    if rem:
        pltpu.sync_copy(zsrc.at[pl.ds(0, rem)],
                        acc_msg.at[pl.ds((s + 1) * rows_per_tile - rem, rem)])
    plsc.subcore_barrier()

    def start_idx(t, b):
        pltpu.async_copy(edges_pk.at[s, t], ed.at[b], sem_i[b])

    def wait_scatters(b):
        for g in range(NG):
            pltpu.make_async_copy(rows.at[b, pl.ds(g * 16, 16)],
                                  acc_msg.at[dstv.at[b, g]], sem_s[b]).wait()

    def mid(t, b, wait_scatter):
        if wait_scatter:
            wait_scatters(b)
        pltpu.make_async_copy(edges_pk.at[s, t], ed.at[b], sem_i[b]).wait()

        def mkidx(g, _):
            sv = ed[b, pl.ds(g * 16, 16)]
            dv = ed[b, pl.ds(KE + g * 16, 16)]
            gidx[b, pl.ds(g * 16, 16)] = sv + c * N
            didx[b, pl.ds(g * 16, 16)] = dv + c * NT
            dstv[b, g, pl.ds(0, 16)] = dv
            return 0

        lax.fori_loop(0, NG, mkidx, 0)
        pltpu.async_copy(h2cat.at[gidx.at[b]], rows.at[b], sem_g[b])
        pltpu.async_copy(adst.at[didx.at[b]], adr.at[b], sem_a[b])

    def finish(t, b):
        pltpu.make_async_copy(h2cat.at[gidx.at[b]], rows.at[b], sem_g[b]).wait()
        pltpu.make_async_copy(adst.at[didx.at[b]], adr.at[b], sem_a[b]).wait()
        lane = lax.iota(jnp.int32, 16)

        def grp(g, _):
            rowi = g * 16 + lane
            c128 = jnp.full((16,), 128, jnp.int32)
            a0 = (plsc.load_gather(rows.at[b], [rowi, c128])
                  + plsc.load_gather(adr.at[b], [rowi, c128 - 128]))
            a1 = (plsc.load_gather(rows.at[b], [rowi, c128 + 1])
                  + plsc.load_gather(adr.at[b], [rowi, c128 - 127]))
            a0 = jnp.where(a0 >= 0.0, a0, 0.2 * a0)
            a1 = jnp.where(a1 >= 0.0, a1, 0.2 * a1)
            e0 = jnp.exp(a0)
            e1 = jnp.exp(a1)
            plsc.store_scatter(rows.at[b], [rowi, c128 + 2], e0)
            plsc.store_scatter(rows.at[b], [rowi, c128 + 3], e1)
            for lj in range(16):
                s0 = e0[lj]
                s1 = e1[lj]
                e = g * 16 + lj
                for j in range(4):
                    rows[b, e, pl.ds(j * 16, 16)] = (
                        s0 * rows[b, e, pl.ds(j * 16, 16)])
                    o = 64 + j * 16
                    rows[b, e, pl.ds(o, 16)] = s1 * rows[b, e, pl.ds(o, 16)]
            pltpu.async_copy(rows.at[b, pl.ds(g * 16, 16)],
                             acc_msg.at[dstv.at[b, g]], sem_s[b], add=True)
            return 0

        lax.fori_loop(0, NG, grp, 0)

    start_idx(0, 0)
    start_idx(1, 1)
    mid(0, 0, wait_scatter=False)
    mid(1, 1, wait_scatter=False)

    def pipe(k, _):
        t0 = 2 * k

        @pl.when(t0 + 2 < NCH)
        def _():
            start_idx(t0 + 2, 0)
            start_idx(t0 + 3, 1)

        finish(t0, 0)

        @pl.when(t0 + 2 < NCH)
        def _():
            mid(t0 + 2, 0, wait_scatter=True)

        finish(t0 + 1, 1)

        @pl.when(t0 + 3 < NCH)
        def _():
            mid(t0 + 3, 1, wait_scatter=True)

        return 0

    lax.fori_loop(0, NCH // 2, pipe, 0)
    wait_scatters(0)
    wait_scatters(1)
    plsc.subcore_barrier()

    pltpu.sync_copy(acc_msg.at[pl.ds(s * rows_per_tile, rows_per_tile)],
                    out_msg.at[c, pl.ds(s * rows_per_tile, rows_per_tile)])
    return


def _run_gat_sc(h2cat, adst, edges_pk):
    mesh = plsc.VectorSubcoreMesh(core_axis_name="c", subcore_axis_name="s",
                                  num_cores=NCORE, num_subcores=NSUB)
    f = pl.kernel(
        _sc_body,
        out_type=[
            jax.ShapeDtypeStruct((NCORE, NPAD, 144), jnp.float32),
        ],
        mesh=mesh,
        compiler_params=pltpu.CompilerParams(needs_layout_passes=False,
                                             use_tc_tiling_on_sc=False),
        scratch_types=[
            pltpu.VMEM_SHARED((NPAD, 144), jnp.float32),
            pltpu.VMEM((2, 2 * KE), jnp.int32),
            pltpu.VMEM((2, KE), jnp.int32),
            pltpu.VMEM((2, KE), jnp.int32),
            pltpu.VMEM((2, NG, 16), jnp.int32),
            pltpu.VMEM((2, KE, 144), jnp.float32),
            pltpu.VMEM((2, KE, 16), jnp.float32),
            pltpu.SemaphoreType.DMA,
            pltpu.SemaphoreType.DMA,
            pltpu.SemaphoreType.DMA,
            pltpu.SemaphoreType.DMA,
            pltpu.SemaphoreType.DMA,
            pltpu.SemaphoreType.DMA,
            pltpu.SemaphoreType.DMA,
            pltpu.SemaphoreType.DMA,
        ],
    )
    (out,) = f(h2cat, adst, edges_pk)
    return out


def _ln(y, g, b):
    mu = jnp.mean(y, axis=-1, keepdims=True)
    var = jnp.mean((y - mu) ** 2, axis=-1, keepdims=True)
    return (y - mu) / jnp.sqrt(var + 1e-5) * g + b


def _post_body(mbu_ref, mtd_ref, bbu_ref, gbu_ref, bebu_ref,
               btd_ref, gtd_ref, betd_ref, wfut_ref, bfu_ref, go_ref, beo_ref,
               o_ref):
    outs = []
    for m_ref, b_ref, g_ref, be_ref in (
        (mbu_ref, bbu_ref, gbu_ref, bebu_ref),
        (mtd_ref, btd_ref, gtd_ref, betd_ref),
    ):
        cols = []
        for core in range(NCORE):
            blkc = m_ref[core]
            for hh in range(2):
                d1 = blkc[:, 130 + hh:131 + hh] + 1e-16
                cols.append(blkc[:, hh * 64:(hh + 1) * 64] / d1)
        y = jnp.concatenate(cols, axis=-1) + b_ref[...]
        outs.append(jnp.maximum(_ln(y, g_ref[...], be_ref[...]), 0.0))
    fused = jnp.concatenate(outs, axis=-1)
    y = jnp.dot(fused, wfut_ref[...]) + bfu_ref[...]
    y = _ln(y, go_ref[...], beo_ref[...])
    nrm = jnp.sqrt(jnp.sum(y * y, axis=-1, keepdims=True))
    o_ref[...] = y / jnp.maximum(nrm, 1e-12)


def _run_post(mbu, mtd, b_bu, g_bu, be_bu, b_td, g_td, be_td,
              WfuT, b_fu, g_out, be_out):
    grid = (N // BLK,)
    wspec = lambda shape: pl.BlockSpec(shape, lambda i: tuple(0 for _ in shape))
    mspec = pl.BlockSpec((NCORE, BLK, 144), lambda i: (0, i, 0))
    return pl.pallas_call(
        _post_body,
        grid=grid,
        in_specs=[
            mspec, mspec,
            wspec((HID,)), wspec((HID,)), wspec((HID,)),
            wspec((HID,)), wspec((HID,)), wspec((HID,)),
            wspec((2 * HID, OUT)), wspec((OUT,)), wspec((OUT,)), wspec((OUT,)),
        ],
        out_specs=pl.BlockSpec((BLK, OUT), lambda i: (i, 0)),
        out_shape=jax.ShapeDtypeStruct((N, OUT), jnp.float32),
    )(mbu, mtd, b_bu, g_bu, be_bu, b_td, g_td, be_td,
      WfuT, b_fu, g_out, be_out)


def _pad_edges(edge_index):
    npad = EPAD - (edge_index.shape[1] + N)
    loop = jnp.arange(N, dtype=jnp.int32)
    src = jnp.concatenate([edge_index[0], loop,
                           jnp.zeros((npad,), jnp.int32)])
    dst = jnp.concatenate([edge_index[1], loop,
                           jnp.full((npad,), DUMMY, jnp.int32)])
    srcr = src.reshape(NSUB, NCH, 1, KE)
    dstr = dst.reshape(NSUB, NCH, 1, KE)
    return jnp.concatenate([srcr, dstr], axis=2).reshape(NSUB, NCH, 2 * KE)


def kernel(x, edge_index_bu, edge_index_td, W_in, b_in, W_bu, att_src_bu,
           att_dst_bu, b_bu, g_bu, be_bu, W_td, att_src_td, att_dst_td, b_td,
           g_td, be_td, W_fu, b_fu, g_out, be_out):
    m = jnp.repeat(jnp.eye(H, dtype=jnp.float32), C, axis=0)
    h2bu, h2td, abu_d, atd_d = _run_pre(
        x, W_in.T, b_in, W_bu.T, att_src_bu.reshape(-1), att_dst_bu.reshape(-1),
        W_td.T, att_src_td.reshape(-1), att_dst_td.reshape(-1), m)

    epk_bu = _pad_edges(edge_index_bu)
    epk_td = _pad_edges(edge_index_td)

    mbu = _run_gat_sc(h2bu.reshape(NCORE * N, 144),
                      abu_d.reshape(NCORE * NT, 16), epk_bu)
    mtd = _run_gat_sc(h2td.reshape(NCORE * N, 144),
                      atd_d.reshape(NCORE * NT, 16), epk_td)

    return _run_post(mbu, mtd, b_bu, g_bu, be_bu, b_td, g_td, be_td,
                     W_fu.T, b_fu, g_out, be_out)


# speedup vs baseline: 1.4697x; 1.0001x over previous
import jax
import jax.numpy as jnp
from jax import lax
from jax.experimental import pallas as pl
from jax.experimental.pallas import tpu as pltpu
from jax.experimental.pallas import tpu_sc as plsc

N = 10000
IN = 128
HID = 256
OUT = 128
H = 4
C = HID // H

NT = 10016
NPAD = 10240
KE = 112
NSUB = 16
NCORE = 2
NCH = 186
PER_TEC = NCH * KE
EPAD = PER_TEC * NSUB
NG = KE // 16
BLK = 1000
DUMMY = N


def _pre_body(x_ref, wint_ref, bin_ref, wbut_ref, asb_ref, adb_ref,
              wtdt_ref, ast_ref, adt_ref, m_ref,
              h2bu_ref, h2td_ref, abu_d_ref, atd_d_ref):
    h = jnp.maximum(
        jnp.dot(x_ref[...], wint_ref[...]) + bin_ref[...],
        0.0)
    m = m_ref[...]
    z14 = jnp.zeros((h.shape[0], 14), jnp.float32)
    for (wt, a_s, a_d, h2_ref, d_ref) in (
        (wbut_ref, asb_ref, adb_ref, h2bu_ref, abu_d_ref),
        (wtdt_ref, ast_ref, adt_ref, h2td_ref, atd_d_ref),
    ):
        h2 = jnp.dot(h, wt[...])
        a_src = jnp.dot(h2 * a_s[...], m)
        a_dst = jnp.dot(h2 * a_d[...], m)
        h2_ref[0] = jnp.concatenate([h2[:, :128], a_src[:, 0:2], z14], axis=-1)
        h2_ref[1] = jnp.concatenate([h2[:, 128:], a_src[:, 2:4], z14], axis=-1)
        d_ref[0] = jnp.concatenate([a_dst[:, 0:2], z14], axis=-1)
        d_ref[1] = jnp.concatenate([a_dst[:, 2:4], z14], axis=-1)


def _run_pre(x, WinT, b_in, WbuT, asb, adb, WtdT, ast, adt, m):
    grid = (N // BLK,)
    wspec = lambda shape: pl.BlockSpec(shape, lambda i: tuple(0 for _ in shape))
    return pl.pallas_call(
        _pre_body,
        grid=grid,
        in_specs=[
            pl.BlockSpec((BLK, IN), lambda i: (i, 0)),
            wspec((IN, HID)), wspec((HID,)),
            wspec((HID, HID)), wspec((HID,)), wspec((HID,)),
            wspec((HID, HID)), wspec((HID,)), wspec((HID,)),
            wspec((HID, H)),
        ],
        out_specs=[
            pl.BlockSpec((NCORE, BLK, 144), lambda i: (0, i, 0)),
            pl.BlockSpec((NCORE, BLK, 144), lambda i: (0, i, 0)),
            pl.BlockSpec((NCORE, BLK, 16), lambda i: (0, i, 0)),
            pl.BlockSpec((NCORE, BLK, 16), lambda i: (0, i, 0)),
        ],
        out_shape=[
            jax.ShapeDtypeStruct((NCORE, N, 144), jnp.float32),
            jax.ShapeDtypeStruct((NCORE, N, 144), jnp.float32),
            jax.ShapeDtypeStruct((NCORE, NT, 16), jnp.float32),
            jax.ShapeDtypeStruct((NCORE, NT, 16), jnp.float32),
        ],
    )(x, WinT, b_in, WbuT, asb, adb, WtdT, ast, adt, m)


def _sc_body(h2cat, adst, edges_pk, out_msg,
             acc_msg, ed, gidx, didx, dstv, rows, adr,
             sem_i0, sem_i1, sem_g0, sem_g1, sem_a0, sem_a1, sem_s0, sem_s1):
    c = lax.axis_index("c")
    s = lax.axis_index("s")
    sem_i = (sem_i0, sem_i1)
    sem_g = (sem_g0, sem_g1)
    sem_a = (sem_a0, sem_a1)
    sem_s = (sem_s0, sem_s1)

    def zrow(i, _):
        for j in range(9):
            rows[0, i, pl.ds(j * 16, 16)] = jnp.zeros((16,), jnp.float32)
        return 0

    lax.fori_loop(0, KE, zrow, 0)
    rows_per_tile = NPAD // NSUB
    zsrc = rows.at[0]
    for k in range(rows_per_tile // KE):
        pltpu.sync_copy(zsrc, acc_msg.at[pl.ds(s * rows_per_tile + k * KE, KE)])
    rem = rows_per_tile - (rows_per_tile // KE) * KE
    if rem:
        pltpu.sync_copy(zsrc.at[pl.ds(0, rem)],
                        acc_msg.at[pl.ds((s + 1) * rows_per_tile - rem, rem)])
    plsc.subcore_barrier()

    def start_idx(t, b):
        pltpu.async_copy(edges_pk.at[s, t], ed.at[b], sem_i[b])

    def wait_scatters(b):
        for g in range(NG):
            pltpu.make_async_copy(rows.at[b, pl.ds(g * 16, 16)],
                                  acc_msg.at[dstv.at[b, g]], sem_s[b]).wait()

    def mid(t, b, wait_scatter):
        if wait_scatter:
            wait_scatters(b)
        pltpu.make_async_copy(edges_pk.at[s, t], ed.at[b], sem_i[b]).wait()

        def mkidx(g, _):
            sv = ed[b, pl.ds(g * 16, 16)]
            dv = ed[b, pl.ds(KE + g * 16, 16)]
            gidx[b, pl.ds(g * 16, 16)] = sv + c * N
            didx[b, pl.ds(g * 16, 16)] = dv + c * NT
            dstv[b, g, pl.ds(0, 16)] = dv
            return 0

        lax.fori_loop(0, NG, mkidx, 0)
        pltpu.async_copy(h2cat.at[gidx.at[b]], rows.at[b], sem_g[b])
        pltpu.async_copy(adst.at[didx.at[b]], adr.at[b], sem_a[b])

    def finish(t, b):
        pltpu.make_async_copy(h2cat.at[gidx.at[b]], rows.at[b], sem_g[b]).wait()
        pltpu.make_async_copy(adst.at[didx.at[b]], adr.at[b], sem_a[b]).wait()
        lane = lax.iota(jnp.int32, 16)

        def grp(g, _):
            rowi = g * 16 + lane
            c128 = jnp.full((16,), 128, jnp.int32)
            a0 = (plsc.load_gather(rows.at[b], [rowi, c128])
                  + plsc.load_gather(adr.at[b], [rowi, c128 - 128]))
            a1 = (plsc.load_gather(rows.at[b], [rowi, c128 + 1])
                  + plsc.load_gather(adr.at[b], [rowi, c128 - 127]))
            a0 = jnp.where(a0 >= 0.0, a0, 0.2 * a0)
            a1 = jnp.where(a1 >= 0.0, a1, 0.2 * a1)
            e0 = jnp.exp(a0)
            e1 = jnp.exp(a1)
            plsc.store_scatter(rows.at[b], [rowi, c128 + 2], e0)
            plsc.store_scatter(rows.at[b], [rowi, c128 + 3], e1)
            for lj in range(16):
                s0 = e0[lj]
                s1 = e1[lj]
                e = g * 16 + lj
                for j in range(4):
                    rows[b, e, pl.ds(j * 16, 16)] = (
                        s0 * rows[b, e, pl.ds(j * 16, 16)])
                    o = 64 + j * 16
                    rows[b, e, pl.ds(o, 16)] = s1 * rows[b, e, pl.ds(o, 16)]
            pltpu.async_copy(rows.at[b, pl.ds(g * 16, 16)],
                             acc_msg.at[dstv.at[b, g]], sem_s[b], add=True)
            return 0

        lax.fori_loop(0, NG, grp, 0)

    start_idx(0, 0)
    start_idx(1, 1)
    mid(0, 0, wait_scatter=False)
    mid(1, 1, wait_scatter=False)

    def pipe(k, _):
        t0 = 2 * k

        @pl.when(t0 + 2 < NCH)
        def _():
            start_idx(t0 + 2, 0)
            start_idx(t0 + 3, 1)

        finish(t0, 0)

        @pl.when(t0 + 2 < NCH)
        def _():
            mid(t0 + 2, 0, wait_scatter=True)

        finish(t0 + 1, 1)

        @pl.when(t0 + 3 < NCH)
        def _():
            mid(t0 + 3, 1, wait_scatter=True)

        return 0

    lax.fori_loop(0, NCH // 2, pipe, 0)
    wait_scatters(0)
    wait_scatters(1)
    plsc.subcore_barrier()

    pltpu.sync_copy(acc_msg.at[pl.ds(s * rows_per_tile, rows_per_tile)],
                    out_msg.at[c, pl.ds(s * rows_per_tile, rows_per_tile)])
    return


def _run_gat_sc(h2cat, adst, edges_pk):
    mesh = plsc.VectorSubcoreMesh(core_axis_name="c", subcore_axis_name="s",
                                  num_cores=NCORE, num_subcores=NSUB)
    f = pl.kernel(
        _sc_body,
        out_type=[
            jax.ShapeDtypeStruct((NCORE, NPAD, 144), jnp.float32),
        ],
        mesh=mesh,
        compiler_params=pltpu.CompilerParams(needs_layout_passes=False,
                                             use_tc_tiling_on_sc=False),
        scratch_types=[
            pltpu.VMEM_SHARED((NPAD, 144), jnp.float32),
            pltpu.VMEM((2, 2 * KE), jnp.int32),
            pltpu.VMEM((2, KE), jnp.int32),
            pltpu.VMEM((2, KE), jnp.int32),
            pltpu.VMEM((2, NG, 16), jnp.int32),
            pltpu.VMEM((2, KE, 144), jnp.float32),
            pltpu.VMEM((2, KE, 16), jnp.float32),
            pltpu.SemaphoreType.DMA,
            pltpu.SemaphoreType.DMA,
            pltpu.SemaphoreType.DMA,
            pltpu.SemaphoreType.DMA,
            pltpu.SemaphoreType.DMA,
            pltpu.SemaphoreType.DMA,
            pltpu.SemaphoreType.DMA,
            pltpu.SemaphoreType.DMA,
        ],
    )
    (out,) = f(h2cat, adst, edges_pk)
    return out


def _ln(y, g, b):
    mu = jnp.mean(y, axis=-1, keepdims=True)
    var = jnp.mean((y - mu) ** 2, axis=-1, keepdims=True)
    return (y - mu) / jnp.sqrt(var + 1e-5) * g + b


def _post_body(mbu_ref, mtd_ref, bbu_ref, gbu_ref, bebu_ref,
               btd_ref, gtd_ref, betd_ref, wfut_ref, bfu_ref, go_ref, beo_ref,
               o_ref):
    outs = []
    for m_ref, b_ref, g_ref, be_ref in (
        (mbu_ref, bbu_ref, gbu_ref, bebu_ref),
        (mtd_ref, btd_ref, gtd_ref, betd_ref),
    ):
        cols = []
        for core in range(NCORE):
            blkc = m_ref[core]
            for hh in range(2):
                d1 = blkc[:, 130 + hh:131 + hh] + 1e-16
                cols.append(blkc[:, hh * 64:(hh + 1) * 64] / d1)
        y = jnp.concatenate(cols, axis=-1) + b_ref[...]
        outs.append(jnp.maximum(_ln(y, g_ref[...], be_ref[...]), 0.0))
    fused = jnp.concatenate(outs, axis=-1)
    y = jnp.dot(fused, wfut_ref[...]) + bfu_ref[...]
    y = _ln(y, go_ref[...], beo_ref[...])
    nrm = jnp.sqrt(jnp.sum(y * y, axis=-1, keepdims=True))
    o_ref[...] = y / jnp.maximum(nrm, 1e-12)


def _run_post(mbu, mtd, b_bu, g_bu, be_bu, b_td, g_td, be_td,
              WfuT, b_fu, g_out, be_out):
    grid = (N // BLK,)
    wspec = lambda shape: pl.BlockSpec(shape, lambda i: tuple(0 for _ in shape))
    mspec = pl.BlockSpec((NCORE, BLK, 144), lambda i: (0, i, 0))
    return pl.pallas_call(
        _post_body,
        grid=grid,
        in_specs=[
            mspec, mspec,
            wspec((HID,)), wspec((HID,)), wspec((HID,)),
            wspec((HID,)), wspec((HID,)), wspec((HID,)),
            wspec((2 * HID, OUT)), wspec((OUT,)), wspec((OUT,)), wspec((OUT,)),
        ],
        out_specs=pl.BlockSpec((BLK, OUT), lambda i: (i, 0)),
        out_shape=jax.ShapeDtypeStruct((N, OUT), jnp.float32),
    )(mbu, mtd, b_bu, g_bu, be_bu, b_td, g_td, be_td,
      WfuT, b_fu, g_out, be_out)


def _pad_edges(edge_index):
    npad = EPAD - (edge_index.shape[1] + N)
    loop = jnp.arange(N, dtype=jnp.int32)
    src = jnp.concatenate([edge_index[0], loop,
                           jnp.zeros((npad,), jnp.int32)])
    dst = jnp.concatenate([edge_index[1], loop,
                           jnp.full((npad,), DUMMY, jnp.int32)])
    srcr = src.reshape(NSUB, NCH, 1, KE)
    dstr = dst.reshape(NSUB, NCH, 1, KE)
    return jnp.concatenate([srcr, dstr], axis=2).reshape(NSUB, NCH, 2 * KE)


def kernel(x, edge_index_bu, edge_index_td, W_in, b_in, W_bu, att_src_bu,
           att_dst_bu, b_bu, g_bu, be_bu, W_td, att_src_td, att_dst_td, b_td,
           g_td, be_td, W_fu, b_fu, g_out, be_out):
    m = jnp.repeat(jnp.eye(H, dtype=jnp.float32), C, axis=0)
    h2bu, h2td, abu_d, atd_d = _run_pre(
        x, W_in.T, b_in, W_bu.T, att_src_bu.reshape(-1), att_dst_bu.reshape(-1),
        W_td.T, att_src_td.reshape(-1), att_dst_td.reshape(-1), m)

    epk_bu = _pad_edges(edge_index_bu)
    epk_td = _pad_edges(edge_index_td)

    mbu = _run_gat_sc(h2bu.reshape(NCORE * N, 144),
                      abu_d.reshape(NCORE * NT, 16), epk_bu)
    mtd = _run_gat_sc(h2td.reshape(NCORE * N, 144),
                      atd_d.reshape(NCORE * NT, 16), epk_td)

    return _run_post(mbu, mtd, b_bu, g_bu, be_bu, b_td, g_td, be_td,
                     W_fu.T, b_fu, g_out, be_out)
